# Initial kernel scaffold; baseline (speedup 1.0000x reference)
import functools

import jax
import jax.numpy as jnp
from jax import lax
from jax.experimental import pallas as pl
from jax.experimental.pallas import tpu as pltpu
from jax.experimental.pallas import tpu_sc as plsc

_N = 10000
_E = 640000
_H = 64
_NG = 64
_GW = 400
_EB = 512
_NS = 16
_NC = 2
_RPS = _N // _NS


def _sc_mesh():
    return plsc.VectorSubcoreMesh(core_axis_name="core", subcore_axis_name="subcore")


def _sc_gather(h, src2, dst2):
    out_t = (jax.ShapeDtypeStruct((_E, _H), jnp.float32),
             jax.ShapeDtypeStruct((_E, _H), jnp.float32))

    @functools.partial(pl.kernel, out_type=out_t, mesh=_sc_mesh())
    def k(h_hbm, s_hbm, d_hbm, os_hbm, od_hbm):
        def body(s_v, d_v, os_v, od_v):
            pltpu.sync_copy(h_hbm.at[s_v.at[0]], os_v)
            pltpu.sync_copy(h_hbm.at[d_v.at[0]], od_v)

        pltpu.emit_pipeline(
            body,
            grid=(_E // _GW,),
            in_specs=[pl.BlockSpec((1, _GW), lambda i: (0, i)),
                      pl.BlockSpec((1, _GW), lambda i: (0, i))],
            out_specs=[pl.BlockSpec((_GW, _H), lambda i: (i, 0)),
                       pl.BlockSpec((_GW, _H), lambda i: (i, 0))],
            core_axis_name=("core", "subcore"),
            dimension_semantics=(pltpu.PARALLEL,),
        )(s_hbm, d_hbm, os_hbm, od_hbm)

    return k(h, src2, dst2)


def _sc_scatter_add(g, src2):
    @functools.partial(
        pl.kernel, mesh=_sc_mesh(),
        out_type=jax.ShapeDtypeStruct((_NC, _N, _H), jnp.float32),
        scratch_types=[pltpu.VMEM_SHARED((_N, _H), jnp.float32),
                       pltpu.VMEM((_RPS, _H), jnp.float32)],
    )
    def k(g_hbm, i_hbm, o_hbm, acc_sh, zb):
        cid = lax.axis_index("core")
        sid = lax.axis_index("subcore")

        @pl.loop(0, _RPS)
        def _(r):
            for c in range(_H // 16):
                zb[r, pl.ds(c * 16, 16)] = jnp.zeros((16,), jnp.float32)

        pltpu.sync_copy(zb, acc_sh.at[pl.ds(sid * _RPS, _RPS)])
        plsc.subcore_barrier()

        def body(g_v, i_v):
            pltpu.sync_copy(g_v, acc_sh.at[i_v.at[0]], add=True)

        pltpu.emit_pipeline(
            body,
            grid=(_E // _GW,),
            in_specs=[pl.BlockSpec((_GW, _H), lambda i: (i, 0)),
                      pl.BlockSpec((1, _GW), lambda i: (0, i))],
            out_specs=[],
            core_axis_name=("core", "subcore"),
            dimension_semantics=(pltpu.PARALLEL,),
        )(g_hbm, i_hbm)

        plsc.subcore_barrier()
        pltpu.sync_copy(acc_sh.at[pl.ds(sid * _RPS, _RPS)],
                        o_hbm.at[cid, pl.ds(sid * _RPS, _RPS)])

    return k(g, src2)


def _tc_embed(x, We, be):
    def body(x_ref, w_ref, b_ref, o_ref):
        o_ref[...] = jnp.dot(x_ref[...], w_ref[...],
                             preferred_element_type=jnp.float32) + b_ref[...]

    return pl.pallas_call(
        body,
        out_shape=jax.ShapeDtypeStruct((_N, _H), jnp.float32),
    )(x, We, be.reshape(1, _H))


def _tc_message_stats(ghs, ghd, ea, W1, W3, W2, b):
    nb = _E // _EB

    def body(gs_ref, gd_ref, ea_ref, w1_ref, w3_ref, w2_ref, b_ref,
             m_ref, st_ref, acc):
        i = pl.program_id(0)

        @pl.when(i == 0)
        def _():
            acc[...] = jnp.zeros_like(acc)

        m = (jnp.dot(gs_ref[...], w1_ref[...], preferred_element_type=jnp.float32)
             + jnp.dot(ea_ref[...], w2_ref[...], preferred_element_type=jnp.float32)
             + jnp.dot(gd_ref[...], w3_ref[...], preferred_element_type=jnp.float32)
             + b_ref[...])
        m_ref[...] = m
        acc[0:1] += jnp.sum(m, axis=0, keepdims=True)
        acc[1:2] += jnp.sum(m * m, axis=0, keepdims=True)

        @pl.when(i == nb - 1)
        def _():
            st_ref[...] = acc[...]

    return pl.pallas_call(
        body,
        grid=(nb,),
        in_specs=[
            pl.BlockSpec((_EB, _H), lambda i: (i, 0)),
            pl.BlockSpec((_EB, _H), lambda i: (i, 0)),
            pl.BlockSpec((_EB, 16), lambda i: (i, 0)),
            pl.BlockSpec((_H, 2 * _H), lambda i: (0, 0)),
            pl.BlockSpec((_H, 2 * _H), lambda i: (0, 0)),
            pl.BlockSpec((16, 2 * _H), lambda i: (0, 0)),
            pl.BlockSpec((1, 2 * _H), lambda i: (0, 0)),
        ],
        out_specs=[
            pl.BlockSpec((_EB, 2 * _H), lambda i: (i, 0)),
            pl.BlockSpec((2, 2 * _H), lambda i: (0, 0)),
        ],
        out_shape=[
            jax.ShapeDtypeStruct((_E, 2 * _H), jnp.float32),
            jax.ShapeDtypeStruct((2, 2 * _H), jnp.float32),
        ],
        scratch_shapes=[pltpu.VMEM((2, 2 * _H), jnp.float32)],
    )(ghs, ghd, ea, W1, W3, W2, b.reshape(1, 2 * _H))


def _tc_gate(m, scale, shift):
    nb = _E // _EB

    def body(m_ref, sc_ref, sh_ref, g_ref):
        mn = m_ref[...] * sc_ref[...] + sh_ref[...]
        g_ref[...] = jax.nn.sigmoid(mn[:, :_H]) * jax.nn.softplus(mn[:, _H:])

    return pl.pallas_call(
        body,
        grid=(nb,),
        in_specs=[
            pl.BlockSpec((_EB, 2 * _H), lambda i: (i, 0)),
            pl.BlockSpec((1, 2 * _H), lambda i: (0, 0)),
            pl.BlockSpec((1, 2 * _H), lambda i: (0, 0)),
        ],
        out_specs=pl.BlockSpec((_EB, _H), lambda i: (i, 0)),
        out_shape=jax.ShapeDtypeStruct((_E, _H), jnp.float32),
    )(m, scale.reshape(1, 2 * _H), shift.reshape(1, 2 * _H))


def _tc_node_update(h, p0, p1, g, b):
    def body(h_ref, p0_ref, p1_ref, g_ref, b_ref, o_ref):
        om = p0_ref[...] + p1_ref[...]
        mu = jnp.mean(om, axis=0, keepdims=True)
        var = jnp.mean((om - mu) ** 2, axis=0, keepdims=True)
        om = (om - mu) * lax.rsqrt(var + 1e-5) * g_ref[...] + b_ref[...]
        o_ref[...] = jax.nn.softplus(h_ref[...] + om)

    return pl.pallas_call(
        body,
        out_shape=jax.ShapeDtypeStruct((_N, _H), jnp.float32),
    )(h, p0, p1, g.reshape(1, _H), b.reshape(1, _H))


def _tc_pool_head(h, batch_row, W_lin, b_lin, wo_row):
    def body(h_ref, bt_ref, wl_ref, bl_ref, wo_ref, o_ref):
        ohT = (lax.broadcasted_iota(jnp.int32, (_NG, _N), 0)
               == bt_ref[...]).astype(jnp.float32)
        sums = jnp.dot(ohT, h_ref[...], preferred_element_type=jnp.float32)
        cnt = jnp.sum(ohT, axis=1, keepdims=True)
        pooled = sums / jnp.maximum(cnt, 1.0)
        hid = jax.nn.softplus(
            jnp.dot(pooled, wl_ref[...], preferred_element_type=jnp.float32)
            + bl_ref[...])
        ov = jnp.sum(hid * wo_ref[...], axis=1, keepdims=True)
        o_ref[...] = jnp.broadcast_to(ov, (_NG, 128))

    return pl.pallas_call(
        body,
        out_shape=jax.ShapeDtypeStruct((_NG, 128), jnp.float32),
    )(h, batch_row, W_lin, b_lin.reshape(1, 128), wo_row)


def kernel(x, edge_index, edge_attr, batch, W_embed, b_embed, conv_W, conv_b,
           bn_i_g, bn_i_b, bn_o_g, bn_o_b, W_lin, b_lin, W_out, b_out):
    ei = edge_index.astype(jnp.int32)
    src2 = ei[0].reshape(1, _E)
    dst2 = ei[1].reshape(1, _E)
    batch_row = batch.astype(jnp.int32).reshape(1, _N)

    h = _tc_embed(x, W_embed, b_embed)
    for i in range(3):
        W = conv_W[i]
        W1, W2, W3 = W[:_H], W[_H:_H + 16], W[_H + 16:]
        ghs, ghd = _sc_gather(h, src2, dst2)
        m, st = _tc_message_stats(ghs, ghd, edge_attr, W1, W3, W2, conv_b[i])
        mean = st[0] / _E
        var = st[1] / _E - mean * mean
        scale = bn_i_g[i] * lax.rsqrt(var + 1e-5)
        shift = bn_i_b[i] - mean * scale
        gated = _tc_gate(m, scale, shift)
        partials = _sc_scatter_add(gated, src2)
        h = _tc_node_update(h, partials[0], partials[1], bn_o_g[i], bn_o_b[i])

    res = _tc_pool_head(h, batch_row, W_lin, b_lin, W_out.reshape(1, 128))
    return res[:, :1] + b_out


# speedup vs baseline: 2.5921x; 2.5921x over previous
import functools

import jax
import jax.numpy as jnp
from jax import lax
from jax.experimental import pallas as pl
from jax.experimental.pallas import tpu as pltpu
from jax.experimental.pallas import tpu_sc as plsc

_N = 10000
_E = 640000
_H = 64
_M = 128
_NG = 64
_C = 256
_NW = 32
_GWS = 128
_EB = 512
_NS = 16
_NC = 2
_NP = 10240
_RPS = _NP // _NS


def _sc_mesh():
    return plsc.VectorSubcoreMesh(core_axis_name="core", subcore_axis_name="subcore")


def _sc_gather(table, idx2):
    nch = _E // _C
    npairs = (nch + 2 * _NW - 1) // (2 * _NW)

    @functools.partial(
        pl.kernel,
        out_type=jax.ShapeDtypeStruct((_E, _M), jnp.float32),
        mesh=_sc_mesh(),
        scratch_types=[pltpu.VMEM((1, _C), jnp.int32),
                       pltpu.VMEM((1, _C), jnp.int32),
                       pltpu.VMEM((2, _C, _M), jnp.float32),
                       pltpu.SemaphoreType.DMA,
                       pltpu.SemaphoreType.DMA])
    def k(t_hbm, i_hbm, o_hbm, idx0_v, idx1_v, rows_v, sem0, sem1):
        wid = lax.axis_index("subcore") * _NC + lax.axis_index("core")

        @pl.loop(0, npairs)
        def _(t):
            c0 = (2 * t) * _NW + wid
            c1 = (2 * t + 1) * _NW + wid

            @pl.when(c0 < nch)
            def _():
                pltpu.sync_copy(i_hbm.at[0, pl.ds(c0 * _C, _C)], idx0_v.at[0])
                cp = pltpu.async_copy(t_hbm.at[idx0_v.at[0]], rows_v.at[0], sem0)

                @pl.when(c1 < nch)
                def _():
                    pltpu.sync_copy(i_hbm.at[0, pl.ds(c1 * _C, _C)], idx1_v.at[0])
                    cp1 = pltpu.async_copy(t_hbm.at[idx1_v.at[0]], rows_v.at[1], sem1)

                cp.wait()
                pltpu.sync_copy(rows_v.at[0], o_hbm.at[pl.ds(c0 * _C, _C)])

                @pl.when(c1 < nch)
                def _():
                    pltpu.make_async_copy(t_hbm.at[idx1_v.at[0]], rows_v.at[1],
                                          sem1).wait()
                    pltpu.sync_copy(rows_v.at[1], o_hbm.at[pl.ds(c1 * _C, _C)])

    return k(table, idx2)


def _sc_scatter_add(g, src2, zrows):
    @functools.partial(
        pl.kernel, mesh=_sc_mesh(),
        out_type=jax.ShapeDtypeStruct((_NC, _NP, _M), jnp.float32),
        scratch_types=[pltpu.VMEM_SHARED((_NP, _M), jnp.float32)],
    )
    def k(g_hbm, i_hbm, z_hbm, o_hbm, acc_sh):
        cid = lax.axis_index("core")
        sid = lax.axis_index("subcore")

        pltpu.sync_copy(z_hbm, acc_sh.at[pl.ds(sid * _RPS, _RPS)])
        plsc.subcore_barrier()

        def body(g_v, i_v):
            pltpu.sync_copy(g_v, acc_sh.at[i_v.at[0]], add=True)

        pltpu.emit_pipeline(
            body,
            grid=(_E // _GWS,),
            in_specs=[pl.BlockSpec((_GWS, _M), lambda i: (i, 0)),
                      pl.BlockSpec((1, _GWS), lambda i: (0, i))],
            out_specs=[],
            core_axis_name=("core", "subcore"),
            dimension_semantics=(pltpu.PARALLEL,),
        )(g_hbm, i_hbm)

        plsc.subcore_barrier()
        pltpu.sync_copy(acc_sh.at[pl.ds(sid * _RPS, _RPS)],
                        o_hbm.at[cid, pl.ds(sid * _RPS, _RPS)])

    return k(g, src2, zrows)


def _tc_embed(x, We, be):
    def body(x_ref, w_ref, b_ref, o_ref):
        o_ref[...] = jnp.dot(x_ref[...], w_ref[...],
                             preferred_element_type=jnp.float32) + b_ref[...]

    return pl.pallas_call(
        body,
        out_shape=jax.ShapeDtypeStruct((_N, _H), jnp.float32),
    )(x, We, be.reshape(1, _H))


def _tc_ab(h, W1, W3, b):
    def body(h_ref, w1_ref, w3_ref, b_ref, a_ref, b2_ref):
        hv = h_ref[...]
        a_ref[...] = jnp.dot(hv, w1_ref[...], preferred_element_type=jnp.float32)
        b2_ref[...] = jnp.dot(hv, w3_ref[...],
                              preferred_element_type=jnp.float32) + b_ref[...]

    return pl.pallas_call(
        body,
        out_shape=[jax.ShapeDtypeStruct((_N, _M), jnp.float32),
                   jax.ShapeDtypeStruct((_N, _M), jnp.float32)],
    )(h, W1, W3, b.reshape(1, _M))


def _tc_message_stats(ga, gb, ea, W2):
    nb = _E // _EB

    def body(ga_ref, gb_ref, ea_ref, w2_ref, m_ref, st_ref, acc):
        i = pl.program_id(0)

        @pl.when(i == 0)
        def _():
            acc[...] = jnp.zeros_like(acc)

        m = (ga_ref[...] + gb_ref[...]
             + jnp.dot(ea_ref[...], w2_ref[...],
                       preferred_element_type=jnp.float32))
        m_ref[...] = m
        acc[0:1] += jnp.sum(m, axis=0, keepdims=True)
        acc[1:2] += jnp.sum(m * m, axis=0, keepdims=True)

        @pl.when(i == nb - 1)
        def _():
            st_ref[...] = acc[...]

    return pl.pallas_call(
        body,
        grid=(nb,),
        in_specs=[
            pl.BlockSpec((_EB, _M), lambda i: (i, 0)),
            pl.BlockSpec((_EB, _M), lambda i: (i, 0)),
            pl.BlockSpec((_EB, 16), lambda i: (i, 0)),
            pl.BlockSpec((16, _M), lambda i: (0, 0)),
        ],
        out_specs=[
            pl.BlockSpec((_EB, _M), lambda i: (i, 0)),
            pl.BlockSpec((2, _M), lambda i: (0, 0)),
        ],
        out_shape=[
            jax.ShapeDtypeStruct((_E, _M), jnp.float32),
            jax.ShapeDtypeStruct((2, _M), jnp.float32),
        ],
        scratch_shapes=[pltpu.VMEM((2, _M), jnp.float32)],
    )(ga, gb, ea, W2)


def _tc_gate(m, scale, shift):
    nb = _E // _EB

    def body(m_ref, sc_ref, sh_ref, g_ref):
        mn = m_ref[...] * sc_ref[...] + sh_ref[...]
        gt = jax.nn.sigmoid(mn[:, :_H]) * jax.nn.softplus(mn[:, _H:])
        g_ref[...] = jnp.concatenate([gt, jnp.zeros_like(gt)], axis=1)

    return pl.pallas_call(
        body,
        grid=(nb,),
        in_specs=[
            pl.BlockSpec((_EB, _M), lambda i: (i, 0)),
            pl.BlockSpec((1, _M), lambda i: (0, 0)),
            pl.BlockSpec((1, _M), lambda i: (0, 0)),
        ],
        out_specs=pl.BlockSpec((_EB, _M), lambda i: (i, 0)),
        out_shape=jax.ShapeDtypeStruct((_E, _M), jnp.float32),
    )(m, scale.reshape(1, _M), shift.reshape(1, _M))


def _tc_node_update(h, p0, p1, g, b):
    def body(h_ref, p0_ref, p1_ref, g_ref, b_ref, o_ref):
        om = p0_ref[...] + p1_ref[...]
        mu = jnp.mean(om, axis=0, keepdims=True)
        var = jnp.mean((om - mu) ** 2, axis=0, keepdims=True)
        om = (om - mu) * lax.rsqrt(var + 1e-5) * g_ref[...] + b_ref[...]
        o_ref[...] = jax.nn.softplus(h_ref[...] + om)

    return pl.pallas_call(
        body,
        out_shape=jax.ShapeDtypeStruct((_N, _H), jnp.float32),
    )(h, p0, p1, g.reshape(1, _H), b.reshape(1, _H))


def _tc_pool_head(h, batch_row, W_lin, b_lin, wo_row):
    def body(h_ref, bt_ref, wl_ref, bl_ref, wo_ref, o_ref):
        ohT = (lax.broadcasted_iota(jnp.int32, (_NG, _N), 0)
               == bt_ref[...]).astype(jnp.float32)
        sums = jnp.dot(ohT, h_ref[...], preferred_element_type=jnp.float32)
        cnt = jnp.sum(ohT, axis=1, keepdims=True)
        pooled = sums / jnp.maximum(cnt, 1.0)
        hid = jax.nn.softplus(
            jnp.dot(pooled, wl_ref[...], preferred_element_type=jnp.float32)
            + bl_ref[...])
        ov = jnp.sum(hid * wo_ref[...], axis=1, keepdims=True)
        o_ref[...] = jnp.broadcast_to(ov, (_NG, 128))

    return pl.pallas_call(
        body,
        out_shape=jax.ShapeDtypeStruct((_NG, 128), jnp.float32),
    )(h, batch_row, W_lin, b_lin.reshape(1, 128), wo_row)


def kernel(x, edge_index, edge_attr, batch, W_embed, b_embed, conv_W, conv_b,
           bn_i_g, bn_i_b, bn_o_g, bn_o_b, W_lin, b_lin, W_out, b_out):
    with jax.enable_x64(False):
        return _run(x, edge_index, edge_attr, batch, W_embed, b_embed,
                    conv_W, conv_b, bn_i_g, bn_i_b, bn_o_g, bn_o_b,
                    W_lin, b_lin, W_out, b_out)


def _run(x, edge_index, edge_attr, batch, W_embed, b_embed, conv_W, conv_b,
         bn_i_g, bn_i_b, bn_o_g, bn_o_b, W_lin, b_lin, W_out, b_out):
    ei = edge_index.astype(jnp.int32)
    src2 = ei[0].reshape(1, _E)
    dst2 = ei[1].reshape(1, _E)
    batch_row = batch.astype(jnp.int32).reshape(1, _N)
    zrows = jnp.zeros((_RPS, _M), jnp.float32)

    h = _tc_embed(x, W_embed, b_embed)
    for i in range(3):
        W = conv_W[i]
        W1, W2, W3 = W[:_H], W[_H:_H + 16], W[_H + 16:]
        A, B = _tc_ab(h, W1, W3, conv_b[i])
        ga = _sc_gather(A, src2)
        gb = _sc_gather(B, dst2)
        m, st = _tc_message_stats(ga, gb, edge_attr, W2)
        mean = st[0] / _E
        var = st[1] / _E - mean * mean
        scale = bn_i_g[i] * lax.rsqrt(var + 1e-5)
        shift = bn_i_b[i] - mean * scale
        gated = _tc_gate(m, scale, shift)
        partials = _sc_scatter_add(gated, src2, zrows)
        h = _tc_node_update(h, partials[0, :_N, :_H], partials[1, :_N, :_H],
                            bn_o_g[i], bn_o_b[i])

    res = _tc_pool_head(h, batch_row, W_lin, b_lin, W_out.reshape(1, 128))
    return res[:, :1] + b_out
